# baseline (speedup 1.0000x reference)
import math

import jax
import jax.numpy as jnp
from jax import lax
from jax.experimental import pallas as pl
from jax.experimental.pallas import tpu as pltpu
from jax.experimental.pallas import tpu_sc as plsc

_D = 32
_SCALE = math.sqrt(float(_D))
_NC, _NS = 2, 16
_NW = _NC * _NS
_G = 128
_CB = 128
_GU = _CB // _G
_L = 16
_NBUF = 8


def _worker(table_hbm, idx_hbm, out_hbm, *scratch):
  idx_v = scratch[0]
  bufs = scratch[1:1 + _NBUF]
  bufTs = scratch[1 + _NBUF:1 + 2 * _NBUF]
  gsems = scratch[1 + 2 * _NBUF:1 + 3 * _NBUF]
  ssems = scratch[1 + 3 * _NBUF:1 + 4 * _NBUF]

  batch = out_hbm.shape[2]
  jb_per_s = batch // _CB
  n_units = idx_v.shape[0] // _GU
  wid = lax.axis_index("s") * _NC + lax.axis_index("c")
  u0 = wid * n_units
  iota = lax.iota(jnp.int32, _L)

  pltpu.sync_copy(idx_hbm.at[wid], idx_v)

  def issue_gathers(t, j):
    for g in range(_GU):
      pltpu.async_copy(table_hbm.at[idx_v.at[t * _GU + g]],
                       bufs[j].at[pl.ds(g * _G, _G)], gsems[j])

  def drain_gathers(j):
    for g in range(_GU):
      pltpu.make_async_copy(table_hbm.at[idx_v.at[0]],
                            bufs[j].at[pl.ds(g * _G, _G)], gsems[j]).wait()

  iota2 = iota + _L

  def transpose_scale(j):
    buf, bufT = bufs[j], bufTs[j]

    @pl.loop(0, _CB // _L)
    def _(k):
      b0 = k * _L
      for bb in range(_L):
        cols = jnp.full((_L,), bb, jnp.int32) + b0
        plsc.store_scatter(bufT, [iota, cols],
                           buf[b0 + bb, pl.ds(0, _L)] * _SCALE)
        plsc.store_scatter(bufT, [iota2, cols],
                           buf[b0 + bb, pl.ds(_L, _L)] * _SCALE)

  def issue_write(t, j):
    u = u0 + t
    s = u // jb_per_s
    jb = u % jb_per_s
    pltpu.async_copy(bufTs[j], out_hbm.at[s, :, pl.ds(jb * _CB, _CB)],
                     ssems[j])

  def wait_write(j):
    pltpu.make_async_copy(bufTs[j], out_hbm.at[0, :, pl.ds(0, _CB)],
                          ssems[j]).wait()

  for j in range(_NBUF - 1):
    issue_gathers(j, j)

  @pl.loop(0, n_units, step=_NBUF)
  def _(q):
    for j in range(_NBUF):
      t = q + j
      drain_gathers(j)
      transpose_scale(j)
      issue_write(t, j)
      jn = (j + _NBUF - 1) % _NBUF

      @pl.when(t + _NBUF - 1 < n_units)
      def _():
        @pl.when(t > 0)
        def _():
          wait_write(jn)

        issue_gathers(t + _NBUF - 1, jn)

  for j in range(_NBUF):
    wait_write(j)


@jax.jit
def kernel(token_tensor, table):
  batch, seq = token_tensor.shape
  n_tok = batch * seq
  n_idx_rows = n_tok // (_NW * _G)
  idx3 = token_tensor.T.astype(jnp.int32).reshape(_NW, n_idx_rows, _G)

  mesh = plsc.VectorSubcoreMesh(core_axis_name="c", subcore_axis_name="s")
  scratch = [pltpu.VMEM((n_idx_rows, _G), jnp.int32)]
  scratch += [pltpu.VMEM((_CB, _D), jnp.float32) for _ in range(_NBUF)]
  scratch += [pltpu.VMEM((_D, _CB), jnp.float32) for _ in range(_NBUF)]
  scratch += [pltpu.SemaphoreType.DMA for _ in range(2 * _NBUF)]
  run = pl.kernel(
      _worker,
      out_type=jax.ShapeDtypeStruct((seq, _D, batch), jnp.float32),
      mesh=mesh,
      compiler_params=pltpu.CompilerParams(use_tc_tiling_on_sc=False,
                                           needs_layout_passes=False),
      scratch_types=scratch,
  )
  out = run(table, idx3)
  return out.transpose(2, 0, 1)


# speedup vs baseline: 1.0175x; 1.0175x over previous
import math

import jax
import jax.numpy as jnp
from jax import lax
from jax.experimental import pallas as pl
from jax.experimental.pallas import tpu as pltpu
from jax.experimental.pallas import tpu_sc as plsc

_D = 32
_SCALE = math.sqrt(float(_D))
_NC, _NS = 2, 16
_NW = _NC * _NS
_G = 128
_CB = 128
_GU = _CB // _G
_L = 16
_NBUF = 8


def _worker(table_hbm, idx_hbm, out_hbm, *scratch):
  idx_v = scratch[0]
  bufs = scratch[1:1 + _NBUF]
  bufTs = scratch[1 + _NBUF:1 + 2 * _NBUF]
  gsems = scratch[1 + 2 * _NBUF:1 + 3 * _NBUF]
  ssems = scratch[1 + 3 * _NBUF:1 + 4 * _NBUF]

  batch = out_hbm.shape[2]
  jb_per_s = batch // _CB
  n_units = idx_v.shape[0] // _GU
  wid = lax.axis_index("s") * _NC + lax.axis_index("c")
  u0 = wid * n_units
  iota = lax.iota(jnp.int32, _L)

  pltpu.sync_copy(idx_hbm.at[wid], idx_v)

  def issue_gathers(t, j):
    for g in range(_GU):
      pltpu.async_copy(table_hbm.at[idx_v.at[t * _GU + g]],
                       bufs[j].at[pl.ds(g * _G, _G)], gsems[j])

  def drain_gathers(j):
    for g in range(_GU):
      pltpu.make_async_copy(table_hbm.at[idx_v.at[0]],
                            bufs[j].at[pl.ds(g * _G, _G)], gsems[j]).wait()

  iota2 = iota + _L

  def transpose_scale(j):
    buf, bufT = bufs[j], bufTs[j]

    @pl.loop(0, _CB // _L)
    def _(k):
      b0 = k * _L
      for bb in range(_L):
        cols = jnp.full((_L,), bb, jnp.int32) + b0
        plsc.store_scatter(bufT, [iota, cols],
                           buf[b0 + bb, pl.ds(0, _L)] * _SCALE)
        plsc.store_scatter(bufT, [iota2, cols],
                           buf[b0 + bb, pl.ds(_L, _L)] * _SCALE)

  def issue_write(t, j):
    u = u0 + t
    s = u // jb_per_s
    jb = u % jb_per_s
    pltpu.async_copy(bufTs[j], out_hbm.at[s, :, pl.ds(jb * _CB, _CB)],
                     ssems[j])

  def wait_write(j):
    pltpu.make_async_copy(bufTs[j], out_hbm.at[0, :, pl.ds(0, _CB)],
                          ssems[j]).wait()

  for j in range(_NBUF - 1):
    issue_gathers(j, j)

  @pl.loop(0, n_units, step=_NBUF)
  def _(q):
    for j in range(_NBUF):
      t = q + j
      drain_gathers(j)
      transpose_scale(j)
      issue_write(t, j)
      jn = (j + _NBUF - 1) % _NBUF

      @pl.when(t + _NBUF - 1 < n_units)
      def _():
        @pl.when(t > 0)
        def _():
          wait_write(jn)

        issue_gathers(t + _NBUF - 1, jn)

  for j in range(_NBUF):
    wait_write(j)


@jax.jit
def kernel(token_tensor, table):
  batch, seq = token_tensor.shape
  n_tok = batch * seq
  n_idx_rows = n_tok // (_NW * _G)
  idx3 = (token_tensor.T.astype(jnp.int32) * 4).reshape(_NW, n_idx_rows, _G)
  table_p = jnp.pad(table, ((0, 0), (0, 96))).reshape(4 * table.shape[0], _D)

  mesh = plsc.VectorSubcoreMesh(core_axis_name="c", subcore_axis_name="s")
  scratch = [pltpu.VMEM((n_idx_rows, _G), jnp.int32)]
  scratch += [pltpu.VMEM((_CB, _D), jnp.float32) for _ in range(_NBUF)]
  scratch += [pltpu.VMEM((_D, _CB), jnp.float32) for _ in range(_NBUF)]
  scratch += [pltpu.SemaphoreType.DMA for _ in range(2 * _NBUF)]
  run = pl.kernel(
      _worker,
      out_type=jax.ShapeDtypeStruct((seq, _D, batch), jnp.float32),
      mesh=mesh,
      compiler_params=pltpu.CompilerParams(use_tc_tiling_on_sc=False,
                                           needs_layout_passes=False),
      scratch_types=scratch,
  )
  out = run(table_p, idx3)
  return out.transpose(2, 0, 1)


# speedup vs baseline: 1.3541x; 1.3309x over previous
import math

import jax
import jax.numpy as jnp
from jax import lax
from jax.experimental import pallas as pl
from jax.experimental.pallas import tpu as pltpu
from jax.experimental.pallas import tpu_sc as plsc

_D = 32
_SCALE = math.sqrt(float(_D))
_NC, _NS = 2, 16
_NW = _NC * _NS
_G = 128
_CB = 128
_GU = _CB // _G
_L = 16
_NBUF = 8
_PT = _CB + 8


def _worker(table_hbm, idx_hbm, out_hbm, *scratch):
  idx_v = scratch[0]
  bufs = scratch[1:1 + _NBUF]
  bufTs = scratch[1 + _NBUF:1 + 2 * _NBUF]
  gsems = scratch[1 + 2 * _NBUF:1 + 3 * _NBUF]
  ssems = scratch[1 + 3 * _NBUF:1 + 4 * _NBUF]

  batch = out_hbm.shape[2]
  jb_per_s = batch // _CB
  n_units = idx_v.shape[0] // _GU
  wid = lax.axis_index("s") * _NC + lax.axis_index("c")
  u0 = wid * n_units
  iota = lax.iota(jnp.int32, _L)

  pltpu.sync_copy(idx_hbm.at[wid], idx_v)

  def issue_gathers(t, j):
    for g in range(_GU):
      pltpu.async_copy(table_hbm.at[idx_v.at[t * _GU + g]],
                       bufs[j].at[pl.ds(g * _G, _G)], gsems[j])

  def drain_gathers(j):
    for g in range(_GU):
      pltpu.make_async_copy(table_hbm.at[idx_v.at[0]],
                            bufs[j].at[pl.ds(g * _G, _G)], gsems[j]).wait()

  iota2 = iota + _L

  def transpose_scale(j):
    buf, bufT = bufs[j], bufTs[j]

    @pl.loop(0, _CB // _L)
    def _(k):
      b0 = k * _L
      for bb in range(_L):
        cols = jnp.full((_L,), bb, jnp.int32) + b0
        plsc.store_scatter(bufT, [iota, cols],
                           buf[b0 + bb, pl.ds(0, _L)] * _SCALE)
        plsc.store_scatter(bufT, [iota2, cols],
                           buf[b0 + bb, pl.ds(_L, _L)] * _SCALE)

  def issue_write(t, j):
    u = u0 + t
    s = u // jb_per_s
    jb = u % jb_per_s
    pltpu.async_copy(bufTs[j].at[:, pl.ds(0, _CB)],
                     out_hbm.at[s, :, pl.ds(jb * _CB, _CB)], ssems[j])

  def wait_write(j):
    pltpu.make_async_copy(bufTs[j].at[:, pl.ds(0, _CB)],
                          out_hbm.at[0, :, pl.ds(0, _CB)], ssems[j]).wait()

  for j in range(_NBUF - 1):
    issue_gathers(j, j)

  @pl.loop(0, n_units, step=_NBUF)
  def _(q):
    for j in range(_NBUF):
      t = q + j
      drain_gathers(j)
      transpose_scale(j)
      issue_write(t, j)
      jn = (j + _NBUF - 1) % _NBUF

      @pl.when(t + _NBUF - 1 < n_units)
      def _():
        @pl.when(t > 0)
        def _():
          wait_write(jn)

        issue_gathers(t + _NBUF - 1, jn)

  for j in range(_NBUF):
    wait_write(j)


@jax.jit
def kernel(token_tensor, table):
  batch, seq = token_tensor.shape
  n_tok = batch * seq
  n_idx_rows = n_tok // (_NW * _G)
  idx3 = (token_tensor.T.astype(jnp.int32) * 4).reshape(_NW, n_idx_rows, _G)
  table_p = jnp.pad(table, ((0, 0), (0, 96))).reshape(4 * table.shape[0], _D)

  mesh = plsc.VectorSubcoreMesh(core_axis_name="c", subcore_axis_name="s")
  scratch = [pltpu.VMEM((n_idx_rows, _G), jnp.int32)]
  scratch += [pltpu.VMEM((_CB, _D), jnp.float32) for _ in range(_NBUF)]
  scratch += [pltpu.VMEM((_D, _PT), jnp.float32) for _ in range(_NBUF)]
  scratch += [pltpu.SemaphoreType.DMA for _ in range(2 * _NBUF)]
  run = pl.kernel(
      _worker,
      out_type=jax.ShapeDtypeStruct((seq, _D, batch), jnp.float32),
      mesh=mesh,
      compiler_params=pltpu.CompilerParams(use_tc_tiling_on_sc=False,
                                           needs_layout_passes=False),
      scratch_types=scratch,
  )
  out = run(table_p, idx3)
  return out.transpose(2, 0, 1)


# speedup vs baseline: 1.3818x; 1.0205x over previous
import math

import jax
import jax.numpy as jnp
from jax import lax
from jax.experimental import pallas as pl
from jax.experimental.pallas import tpu as pltpu
from jax.experimental.pallas import tpu_sc as plsc

_D = 32
_SCALE = math.sqrt(float(_D))
_NC, _NS = 2, 16
_NW = _NC * _NS
_G = 128
_CB = 256
_GU = _CB // _G
_L = 16
_NBUF = 5
_PT = _CB + 8


def _worker(table_hbm, idx_hbm, out_hbm, *scratch):
  idx_v = scratch[0]
  bufs = scratch[1:1 + _NBUF]
  bufTs = scratch[1 + _NBUF:1 + 2 * _NBUF]
  gsems = scratch[1 + 2 * _NBUF:1 + 3 * _NBUF]
  ssems = scratch[1 + 3 * _NBUF:1 + 4 * _NBUF]

  batch = out_hbm.shape[2]
  jb_per_s = batch // _CB
  n_units = idx_v.shape[0] // _GU
  wid = lax.axis_index("s") * _NC + lax.axis_index("c")
  u0 = wid * n_units
  iota = lax.iota(jnp.int32, _L)

  pltpu.sync_copy(idx_hbm.at[wid], idx_v)

  def issue_gathers(t, j):
    for g in range(_GU):
      pltpu.async_copy(table_hbm.at[idx_v.at[t * _GU + g]],
                       bufs[j].at[pl.ds(g * _G, _G)], gsems[j])

  def drain_gathers(j):
    for g in range(_GU):
      pltpu.make_async_copy(table_hbm.at[idx_v.at[0]],
                            bufs[j].at[pl.ds(g * _G, _G)], gsems[j]).wait()

  iota2 = iota + _L

  def transpose_scale(j):
    buf, bufT = bufs[j], bufTs[j]

    @pl.loop(0, _CB // _L)
    def _(k):
      b0 = k * _L
      for bb in range(_L):
        cols = jnp.full((_L,), bb, jnp.int32) + b0
        plsc.store_scatter(bufT, [iota, cols],
                           buf[b0 + bb, pl.ds(0, _L)] * _SCALE)
        plsc.store_scatter(bufT, [iota2, cols],
                           buf[b0 + bb, pl.ds(_L, _L)] * _SCALE)

  def issue_write(t, j):
    u = u0 + t
    s = u // jb_per_s
    jb = u % jb_per_s
    pltpu.async_copy(bufTs[j].at[:, pl.ds(0, _CB)],
                     out_hbm.at[s, :, pl.ds(jb * _CB, _CB)], ssems[j])

  def wait_write(j):
    pltpu.make_async_copy(bufTs[j].at[:, pl.ds(0, _CB)],
                          out_hbm.at[0, :, pl.ds(0, _CB)], ssems[j]).wait()

  for j in range(_NBUF - 1):
    issue_gathers(j, j)

  @pl.loop(0, n_units, step=_NBUF)
  def _(q):
    for j in range(_NBUF):
      t = q + j
      drain_gathers(j)
      transpose_scale(j)
      issue_write(t, j)
      jn = (j + _NBUF - 1) % _NBUF

      @pl.when(t + _NBUF - 1 < n_units)
      def _():
        @pl.when(t > 0)
        def _():
          wait_write(jn)

        issue_gathers(t + _NBUF - 1, jn)

  for j in range(_NBUF):
    wait_write(j)


@jax.jit
def kernel(token_tensor, table):
  batch, seq = token_tensor.shape
  n_tok = batch * seq
  n_idx_rows = n_tok // (_NW * _G)
  idx3 = (token_tensor.T.astype(jnp.int32) * 4).reshape(_NW, n_idx_rows, _G)
  table_p = jnp.pad(table, ((0, 0), (0, 96))).reshape(4 * table.shape[0], _D)

  mesh = plsc.VectorSubcoreMesh(core_axis_name="c", subcore_axis_name="s")
  scratch = [pltpu.VMEM((n_idx_rows, _G), jnp.int32)]
  scratch += [pltpu.VMEM((_CB, _D), jnp.float32) for _ in range(_NBUF)]
  scratch += [pltpu.VMEM((_D, _PT), jnp.float32) for _ in range(_NBUF)]
  scratch += [pltpu.SemaphoreType.DMA for _ in range(2 * _NBUF)]
  run = pl.kernel(
      _worker,
      out_type=jax.ShapeDtypeStruct((seq, _D, batch), jnp.float32),
      mesh=mesh,
      compiler_params=pltpu.CompilerParams(use_tc_tiling_on_sc=False,
                                           needs_layout_passes=False),
      scratch_types=scratch,
  )
  out = run(table_p, idx3)
  return out.transpose(2, 0, 1)
